# baseline (speedup 1.0000x reference)
import functools

import jax
import jax.numpy as jnp
from jax import lax
from jax.experimental import pallas as pl
from jax.experimental.pallas import tpu as pltpu
from jax.experimental.pallas import tpu_sc as plsc

N_NODES = 10000
N_EDGES = 320000
NP = 10112
NC = 2
NS = 16
NW = NC * NS
CH = 64
NCH = 160
CPP = 40
NBUF = 4
EPT = CH * NCH
E_PAD = EPT * NW
ROWS_PT = NP // NS


def _sc_mesh():
  return plsc.VectorSubcoreMesh(core_axis_name="c", subcore_axis_name="s",
                                num_cores=NC, num_subcores=NS)


@functools.partial(
    pl.kernel,
    out_type=(jax.ShapeDtypeStruct((NC * NP,), jnp.float32),
              jax.ShapeDtypeStruct((NC * NP,), jnp.float32)),
    mesh=_sc_mesh(),
    scratch_types=[
        pltpu.VMEM((NCH, CH), jnp.int32),
        pltpu.VMEM((NCH, CH), jnp.int32),
        pltpu.VMEM((CH,), jnp.float32),
        pltpu.VMEM((640,), jnp.float32),
        pltpu.VMEM_SHARED((NP,), jnp.float32),
        pltpu.VMEM_SHARED((NP,), jnp.float32),
    ],
)
def _degree_kernel(src_hbm, dst_hbm, dego_hbm, degi_hbm,
                   src_v, dst_v, ones_v, zbuf_v, dego_sh, degi_sh):
  cid = lax.axis_index("c")
  sid = lax.axis_index("s")
  wid = sid * NC + cid
  row0 = sid * ROWS_PT
  for j in range(640 // 16):
    zbuf_v[pl.ds(j * 16, 16)] = jnp.zeros((16,), jnp.float32)
  for j in range(CH // 16):
    ones_v[pl.ds(j * 16, 16)] = jnp.ones((16,), jnp.float32)
  pltpu.sync_copy(zbuf_v.at[pl.ds(0, ROWS_PT)],
                  dego_sh.at[pl.ds(row0, ROWS_PT)])
  pltpu.sync_copy(zbuf_v.at[pl.ds(0, ROWS_PT)],
                  degi_sh.at[pl.ds(row0, ROWS_PT)])
  pltpu.sync_copy(src_hbm.at[wid], src_v)
  pltpu.sync_copy(dst_hbm.at[wid], dst_v)
  plsc.subcore_barrier()
  def body(j, carry):
    pltpu.sync_copy(ones_v, dego_sh.at[src_v.at[j]], add=True)
    pltpu.sync_copy(ones_v, degi_sh.at[dst_v.at[j]], add=True)
    return carry
  lax.fori_loop(0, NCH, body, 0)
  plsc.subcore_barrier()
  pltpu.sync_copy(dego_sh.at[pl.ds(row0, ROWS_PT)],
                  zbuf_v.at[pl.ds(0, ROWS_PT)])
  pltpu.sync_copy(zbuf_v.at[pl.ds(0, ROWS_PT)],
                  dego_hbm.at[pl.ds(cid * NP + row0, ROWS_PT)])
  pltpu.sync_copy(degi_sh.at[pl.ds(row0, ROWS_PT)],
                  zbuf_v.at[pl.ds(0, ROWS_PT)])
  pltpu.sync_copy(zbuf_v.at[pl.ds(0, ROWS_PT)],
                  degi_hbm.at[pl.ds(cid * NP + row0, ROWS_PT)])


def _make_agg_kernel(d_feat):
  @functools.partial(
      pl.kernel,
      out_type=jax.ShapeDtypeStruct((NC, NP, d_feat), jnp.float32),
      mesh=_sc_mesh(),
      compiler_params=pltpu.CompilerParams(use_tc_tiling_on_sc=False),
      scratch_types=[
          pltpu.VMEM((CPP + NBUF, CH), jnp.int32),
          pltpu.VMEM((CPP, CH), jnp.int32),
          pltpu.VMEM((CH, d_feat), jnp.float32),
          pltpu.VMEM((CH, d_feat), jnp.float32),
          pltpu.VMEM((CH, d_feat), jnp.float32),
          pltpu.VMEM((CH, d_feat), jnp.float32),
          pltpu.VMEM_SHARED((NP, d_feat), jnp.float32),
          pltpu.SemaphoreType.DMA,
      ],
  )
  def agg_kernel(h_hbm, src_hbm, dst_hbm, out_hbm,
                 src_v, dst_v, r0_v, r1_v, r2_v, r3_v, agg_sh, sem):
    bufs = [r0_v, r1_v, r2_v, r3_v]
    cid = lax.axis_index("c")
    sid = lax.axis_index("s")
    wid = sid * NC + cid
    row0 = sid * ROWS_PT
    def zbody(i, carry):
      for j in range(d_feat // 16):
        r0_v[i, pl.ds(j * 16, 16)] = jnp.zeros((16,), jnp.float32)
      return carry
    lax.fori_loop(0, CH, zbody, 0)
    nstep = (ROWS_PT + CH - 1) // CH
    for k in range(nstep):
      sz = CH if k < ROWS_PT // CH else ROWS_PT % CH
      pltpu.sync_copy(r0_v.at[pl.ds(0, sz)],
                      agg_sh.at[pl.ds(row0 + k * CH, sz)])
    for t in range(NBUF):
      for j in range(CH // 16):
        src_v[CPP + t, pl.ds(j * 16, 16)] = jnp.zeros((16,), jnp.int32)
    plsc.subcore_barrier()
    for p in range(NCH // CPP):
      pltpu.sync_copy(src_hbm.at[wid, pl.ds(p * CPP, CPP)],
                      src_v.at[pl.ds(0, CPP)])
      pltpu.sync_copy(dst_hbm.at[wid, pl.ds(p * CPP, CPP)], dst_v)
      for t in range(NBUF - 1):
        pltpu.async_copy(h_hbm.at[src_v.at[t]], bufs[t], sem)
      def body(i, carry):
        j0 = NBUF * i
        for t in range(NBUF):
          pltpu.make_async_copy(h_hbm.at[src_v.at[j0 + t]], bufs[t],
                                sem).wait()
          pltpu.async_copy(h_hbm.at[src_v.at[j0 + t + NBUF - 1]],
                           bufs[(t - 1) % NBUF], sem)
          pltpu.sync_copy(bufs[t], agg_sh.at[dst_v.at[j0 + t]], add=True)
        return carry
      lax.fori_loop(0, CPP // NBUF, body, 0)
      for t in range(NBUF - 1):
        pltpu.make_async_copy(h_hbm.at[src_v.at[CPP + t]],
                              bufs[(CPP + t) % NBUF], sem).wait()
    plsc.subcore_barrier()
    for k in range(nstep):
      sz = CH if k < ROWS_PT // CH else ROWS_PT % CH
      pltpu.sync_copy(agg_sh.at[pl.ds(row0 + k * CH, sz)],
                      r0_v.at[pl.ds(0, sz)])
      pltpu.sync_copy(r0_v.at[pl.ds(0, sz)],
                      out_hbm.at[cid, pl.ds(row0 + k * CH, sz)])
  return agg_kernel


_agg128 = _make_agg_kernel(128)
_agg64 = _make_agg_kernel(64)


def _mm_body(x_ref, w_ref, o_ref):
  o_ref[...] = jnp.dot(x_ref[...], w_ref[...],
                       preferred_element_type=jnp.float32)


def _tc_matmul(x, w):
  return pl.pallas_call(
      _mm_body,
      out_shape=jax.ShapeDtypeStruct((x.shape[0], w.shape[1]), jnp.float32),
  )(x, w)


def _norm_body(hraw_ref, dego_ref, degi_ref, h1_ref, no_ref, ni_ref):
  deg_out = dego_ref[0, :] + dego_ref[1, :]
  deg_in = degi_ref[0, :] + degi_ref[1, :]
  norm_out = lax.rsqrt(jnp.maximum(deg_out, 1.0))
  norm_in = lax.rsqrt(jnp.maximum(deg_in, 1.0))
  no_ref[...] = jnp.broadcast_to(norm_out[:, None], (NP, 128))
  ni_ref[...] = jnp.broadcast_to(norm_in[:, None], (NP, 128))
  h1_ref[...] = hraw_ref[...] * norm_out[:, None]


def _tc_norm_scale(hraw, dego, degi):
  return pl.pallas_call(
      _norm_body,
      out_shape=(
          jax.ShapeDtypeStruct((NP, 128), jnp.float32),
          jax.ShapeDtypeStruct((NP, 128), jnp.float32),
          jax.ShapeDtypeStruct((NP, 128), jnp.float32),
      ),
  )(hraw, dego, degi)


def _mid_body(aggp_ref, ni_ref, no_ref, b1_ref, w2_ref, o_ref):
  agg = aggp_ref[0] + aggp_ref[1]
  h2 = jnp.maximum(agg * ni_ref[...] + b1_ref[...], 0.0)
  o_ref[...] = jnp.dot(h2 * no_ref[...], w2_ref[...],
                       preferred_element_type=jnp.float32)


def _tc_mid(aggp, ni, no, b1, w2):
  return pl.pallas_call(
      _mid_body,
      out_shape=jax.ShapeDtypeStruct((NP, w2.shape[1]), jnp.float32),
  )(aggp, ni, no, b1, w2)


def _final_body(aggp_ref, ni_ref, b2_ref, o_ref):
  agg = aggp_ref[0] + aggp_ref[1]
  o_ref[...] = agg * ni_ref[...] + b2_ref[...]


def _tc_final(aggp, ni, b2):
  return pl.pallas_call(
      _final_body,
      out_shape=jax.ShapeDtypeStruct((NP, b2.shape[1]), jnp.float32),
  )(aggp, ni, b2)


@jax.jit
def kernel(in_feat, edge_index, W1, b1, W2, b2):
  src = edge_index[0]
  dst = edge_index[1]
  pad = E_PAD - N_EDGES
  src_r = jnp.concatenate(
      [src, jnp.full((pad,), N_NODES, jnp.int32)]).reshape(NW, NCH, CH)
  trash = N_NODES + (jnp.arange(pad, dtype=jnp.int32) % (NP - N_NODES))
  dst_r = jnp.concatenate([dst, trash]).reshape(NW, NCH, CH)
  x_pad = jnp.pad(in_feat, ((0, NP - N_NODES), (0, 0)))

  dego, degi = _degree_kernel(src_r, dst_r)
  hraw = _tc_matmul(x_pad, W1)
  h1, no, ni = _tc_norm_scale(hraw, dego.reshape(NC, NP),
                              degi.reshape(NC, NP))
  aggp1 = _agg128(h1, src_r, dst_r)
  h2b = _tc_mid(aggp1, ni, no, b1.reshape(1, 128), W2)
  aggp2 = _agg64(h2b, src_r, dst_r)
  out = _tc_final(aggp2, ni[:, :64], b2.reshape(1, 64))
  return out[:N_NODES]


# speedup vs baseline: 2.2695x; 2.2695x over previous
import functools

import jax
import jax.numpy as jnp
from jax import lax
from jax.experimental import pallas as pl
from jax.experimental.pallas import tpu as pltpu
from jax.experimental.pallas import tpu_sc as plsc

N_NODES = 10000
N_EDGES = 320000
NP = 10112
NC = 2
NS = 16
NW = NC * NS
CH = 128
NCH = 80
CPP = 40
NBUF = 2
EPT = CH * NCH
E_PAD = EPT * NW
ROWS_PT = NP // NS


def _sc_mesh():
  return plsc.VectorSubcoreMesh(core_axis_name="c", subcore_axis_name="s",
                                num_cores=NC, num_subcores=NS)


@functools.partial(
    pl.kernel,
    out_type=(jax.ShapeDtypeStruct((NC * NP,), jnp.float32),
              jax.ShapeDtypeStruct((NC * NP,), jnp.float32)),
    mesh=_sc_mesh(),
    scratch_types=[
        pltpu.VMEM((NCH, CH), jnp.int32),
        pltpu.VMEM((NCH, CH), jnp.int32),
        pltpu.VMEM((CH,), jnp.float32),
        pltpu.VMEM((640,), jnp.float32),
        pltpu.VMEM_SHARED((NP,), jnp.float32),
        pltpu.VMEM_SHARED((NP,), jnp.float32),
    ],
)
def _degree_kernel(src_hbm, dst_hbm, dego_hbm, degi_hbm,
                   src_v, dst_v, ones_v, zbuf_v, dego_sh, degi_sh):
  cid = lax.axis_index("c")
  sid = lax.axis_index("s")
  wid = sid * NC + cid
  row0 = sid * ROWS_PT
  for j in range(640 // 16):
    zbuf_v[pl.ds(j * 16, 16)] = jnp.zeros((16,), jnp.float32)
  for j in range(CH // 16):
    ones_v[pl.ds(j * 16, 16)] = jnp.ones((16,), jnp.float32)
  pltpu.sync_copy(zbuf_v.at[pl.ds(0, ROWS_PT)],
                  dego_sh.at[pl.ds(row0, ROWS_PT)])
  pltpu.sync_copy(zbuf_v.at[pl.ds(0, ROWS_PT)],
                  degi_sh.at[pl.ds(row0, ROWS_PT)])
  pltpu.sync_copy(src_hbm.at[wid], src_v)
  pltpu.sync_copy(dst_hbm.at[wid], dst_v)
  plsc.subcore_barrier()
  def body(j, carry):
    pltpu.sync_copy(ones_v, dego_sh.at[src_v.at[j]], add=True)
    pltpu.sync_copy(ones_v, degi_sh.at[dst_v.at[j]], add=True)
    return carry
  lax.fori_loop(0, NCH, body, 0)
  plsc.subcore_barrier()
  pltpu.sync_copy(dego_sh.at[pl.ds(row0, ROWS_PT)],
                  zbuf_v.at[pl.ds(0, ROWS_PT)])
  pltpu.sync_copy(zbuf_v.at[pl.ds(0, ROWS_PT)],
                  dego_hbm.at[pl.ds(cid * NP + row0, ROWS_PT)])
  pltpu.sync_copy(degi_sh.at[pl.ds(row0, ROWS_PT)],
                  zbuf_v.at[pl.ds(0, ROWS_PT)])
  pltpu.sync_copy(zbuf_v.at[pl.ds(0, ROWS_PT)],
                  degi_hbm.at[pl.ds(cid * NP + row0, ROWS_PT)])


def _make_agg_kernel(d_feat, table_in_spmem=False):
  scratch = [
      pltpu.VMEM((CPP + NBUF, CH), jnp.int32),
      pltpu.VMEM((CPP, CH), jnp.int32),
      pltpu.VMEM((CH, d_feat), jnp.float32),
      pltpu.VMEM((CH, d_feat), jnp.float32),
      pltpu.VMEM_SHARED((NP, d_feat), jnp.float32),
      pltpu.SemaphoreType.DMA,
  ]
  if table_in_spmem:
    scratch.append(pltpu.VMEM_SHARED((NP, d_feat), jnp.float32))

  @functools.partial(
      pl.kernel,
      out_type=jax.ShapeDtypeStruct((NC, NP, d_feat), jnp.float32),
      mesh=_sc_mesh(),
      compiler_params=pltpu.CompilerParams(use_tc_tiling_on_sc=False),
      scratch_types=scratch,
  )
  def agg_kernel(h_hbm, src_hbm, dst_hbm, out_hbm,
                 src_v, dst_v, r0_v, r1_v, agg_sh, sem, *maybe_tab):
    bufs = [r0_v, r1_v]
    cid = lax.axis_index("c")
    sid = lax.axis_index("s")
    wid = sid * NC + cid
    row0 = sid * ROWS_PT
    def zbody(i, carry):
      for j in range(d_feat // 16):
        r0_v[i, pl.ds(j * 16, 16)] = jnp.zeros((16,), jnp.float32)
      return carry
    lax.fori_loop(0, CH, zbody, 0)
    nstep = (ROWS_PT + CH - 1) // CH
    for k in range(nstep):
      sz = CH if k < ROWS_PT // CH else ROWS_PT % CH
      pltpu.sync_copy(r0_v.at[pl.ds(0, sz)],
                      agg_sh.at[pl.ds(row0 + k * CH, sz)])
    if table_in_spmem:
      tab_sh = maybe_tab[0]
      for k in range(nstep):
        sz = CH if k < ROWS_PT // CH else ROWS_PT % CH
        pltpu.sync_copy(h_hbm.at[pl.ds(row0 + k * CH, sz)],
                        r1_v.at[pl.ds(0, sz)])
        pltpu.sync_copy(r1_v.at[pl.ds(0, sz)],
                        tab_sh.at[pl.ds(row0 + k * CH, sz)])
      gsrc = tab_sh
    else:
      gsrc = h_hbm
    for t in range(NBUF):
      for j in range(CH // 16):
        src_v[CPP + t, pl.ds(j * 16, 16)] = jnp.zeros((16,), jnp.int32)
    plsc.subcore_barrier()
    for p in range(NCH // CPP):
      pltpu.sync_copy(src_hbm.at[wid, pl.ds(p * CPP, CPP)],
                      src_v.at[pl.ds(0, CPP)])
      pltpu.sync_copy(dst_hbm.at[wid, pl.ds(p * CPP, CPP)], dst_v)
      for t in range(NBUF - 1):
        pltpu.async_copy(gsrc.at[src_v.at[t]], bufs[t], sem)
      def body(i, carry):
        j0 = NBUF * i
        for t in range(NBUF):
          pltpu.make_async_copy(gsrc.at[src_v.at[j0 + t]], bufs[t],
                                sem).wait()
          pltpu.async_copy(gsrc.at[src_v.at[j0 + t + NBUF - 1]],
                           bufs[(t - 1) % NBUF], sem)
          pltpu.sync_copy(bufs[t], agg_sh.at[dst_v.at[j0 + t]], add=True)
        return carry
      lax.fori_loop(0, CPP // NBUF, body, 0)
      for t in range(NBUF - 1):
        pltpu.make_async_copy(gsrc.at[src_v.at[CPP + t]],
                              bufs[(CPP + t) % NBUF], sem).wait()
    plsc.subcore_barrier()
    for k in range(nstep):
      sz = CH if k < ROWS_PT // CH else ROWS_PT % CH
      pltpu.sync_copy(agg_sh.at[pl.ds(row0 + k * CH, sz)],
                      r0_v.at[pl.ds(0, sz)])
      pltpu.sync_copy(r0_v.at[pl.ds(0, sz)],
                      out_hbm.at[cid, pl.ds(row0 + k * CH, sz)])
  return agg_kernel


_agg128 = _make_agg_kernel(128)
_agg64 = _make_agg_kernel(64, table_in_spmem=True)


def _mm_body(x_ref, w_ref, o_ref):
  o_ref[...] = jnp.dot(x_ref[...], w_ref[...],
                       preferred_element_type=jnp.float32)


def _tc_matmul(x, w):
  return pl.pallas_call(
      _mm_body,
      out_shape=jax.ShapeDtypeStruct((x.shape[0], w.shape[1]), jnp.float32),
  )(x, w)


def _norm_body(hraw_ref, dego_ref, degi_ref, h1_ref, no_ref, ni_ref):
  deg_out = dego_ref[0, :] + dego_ref[1, :]
  deg_in = degi_ref[0, :] + degi_ref[1, :]
  norm_out = lax.rsqrt(jnp.maximum(deg_out, 1.0))
  norm_in = lax.rsqrt(jnp.maximum(deg_in, 1.0))
  no_ref[...] = jnp.broadcast_to(norm_out[:, None], (NP, 128))
  ni_ref[...] = jnp.broadcast_to(norm_in[:, None], (NP, 128))
  h1_ref[...] = hraw_ref[...] * norm_out[:, None]


def _tc_norm_scale(hraw, dego, degi):
  return pl.pallas_call(
      _norm_body,
      out_shape=(
          jax.ShapeDtypeStruct((NP, 128), jnp.float32),
          jax.ShapeDtypeStruct((NP, 128), jnp.float32),
          jax.ShapeDtypeStruct((NP, 128), jnp.float32),
      ),
  )(hraw, dego, degi)


def _mid_body(aggp_ref, ni_ref, no_ref, b1_ref, w2_ref, o_ref):
  agg = aggp_ref[0] + aggp_ref[1]
  h2 = jnp.maximum(agg * ni_ref[...] + b1_ref[...], 0.0)
  o_ref[...] = jnp.dot(h2 * no_ref[...], w2_ref[...],
                       preferred_element_type=jnp.float32)


def _tc_mid(aggp, ni, no, b1, w2):
  return pl.pallas_call(
      _mid_body,
      out_shape=jax.ShapeDtypeStruct((NP, w2.shape[1]), jnp.float32),
  )(aggp, ni, no, b1, w2)


def _final_body(aggp_ref, ni_ref, b2_ref, o_ref):
  agg = aggp_ref[0] + aggp_ref[1]
  o_ref[...] = agg * ni_ref[...] + b2_ref[...]


def _tc_final(aggp, ni, b2):
  return pl.pallas_call(
      _final_body,
      out_shape=jax.ShapeDtypeStruct((NP, b2.shape[1]), jnp.float32),
  )(aggp, ni, b2)


@jax.jit
def kernel(in_feat, edge_index, W1, b1, W2, b2):
  src = edge_index[0]
  dst = edge_index[1]
  pad = E_PAD - N_EDGES
  src_r = jnp.concatenate(
      [src, jnp.full((pad,), N_NODES, jnp.int32)]).reshape(NW, NCH, CH)
  trash = N_NODES + (jnp.arange(pad, dtype=jnp.int32) % (NP - N_NODES))
  dst_r = jnp.concatenate([dst, trash]).reshape(NW, NCH, CH)
  x_pad = jnp.pad(in_feat, ((0, NP - N_NODES), (0, 0)))

  dego, degi = _degree_kernel(src_r, dst_r)
  hraw = _tc_matmul(x_pad, W1)
  h1, no, ni = _tc_norm_scale(hraw, dego.reshape(NC, NP),
                              degi.reshape(NC, NP))
  aggp1 = _agg128(h1, src_r, dst_r)
  h2b = _tc_mid(aggp1, ni, no, b1.reshape(1, 128), W2)
  aggp2 = _agg64(h2b, src_r, dst_r)
  out = _tc_final(aggp2, ni[:, :64], b2.reshape(1, 64))
  return out[:N_NODES]


# speedup vs baseline: 4.5995x; 2.0266x over previous
import functools

import jax
import jax.numpy as jnp
from jax import lax
from jax.experimental import pallas as pl
from jax.experimental.pallas import tpu as pltpu
from jax.experimental.pallas import tpu_sc as plsc

N_NODES = 10000
N_EDGES = 320000
NP = 10112
NC = 2
NS = 16
NW = NC * NS
CH = 128
NCH = 80
CPP = 40
NBUF = 2
EPT = CH * NCH
E_PAD = EPT * NW
ROWS_PT = NP // NS


def _sc_mesh():
  return plsc.VectorSubcoreMesh(core_axis_name="c", subcore_axis_name="s",
                                num_cores=NC, num_subcores=NS)


@functools.partial(
    pl.kernel,
    out_type=(jax.ShapeDtypeStruct((NC * NP,), jnp.float32),
              jax.ShapeDtypeStruct((NC * NP,), jnp.float32)),
    mesh=_sc_mesh(),
    scratch_types=[
        pltpu.VMEM((NCH, CH), jnp.int32),
        pltpu.VMEM((NCH, CH), jnp.int32),
        pltpu.VMEM((CH,), jnp.float32),
        pltpu.VMEM((640,), jnp.float32),
        pltpu.VMEM_SHARED((NP,), jnp.float32),
        pltpu.VMEM_SHARED((NP,), jnp.float32),
    ],
)
def _degree_kernel(src_hbm, dst_hbm, dego_hbm, degi_hbm,
                   src_v, dst_v, ones_v, zbuf_v, dego_sh, degi_sh):
  cid = lax.axis_index("c")
  sid = lax.axis_index("s")
  wid = sid * NC + cid
  row0 = sid * ROWS_PT
  for j in range(640 // 16):
    zbuf_v[pl.ds(j * 16, 16)] = jnp.zeros((16,), jnp.float32)
  for j in range(CH // 16):
    ones_v[pl.ds(j * 16, 16)] = jnp.ones((16,), jnp.float32)
  pltpu.sync_copy(zbuf_v.at[pl.ds(0, ROWS_PT)],
                  dego_sh.at[pl.ds(row0, ROWS_PT)])
  pltpu.sync_copy(zbuf_v.at[pl.ds(0, ROWS_PT)],
                  degi_sh.at[pl.ds(row0, ROWS_PT)])
  pltpu.sync_copy(src_hbm.at[wid], src_v)
  pltpu.sync_copy(dst_hbm.at[wid], dst_v)
  plsc.subcore_barrier()
  def body(j, carry):
    pltpu.sync_copy(ones_v, dego_sh.at[src_v.at[j]], add=True)
    pltpu.sync_copy(ones_v, degi_sh.at[dst_v.at[j]], add=True)
    return carry
  lax.fori_loop(0, NCH, body, 0)
  plsc.subcore_barrier()
  pltpu.sync_copy(dego_sh.at[pl.ds(row0, ROWS_PT)],
                  zbuf_v.at[pl.ds(0, ROWS_PT)])
  pltpu.sync_copy(zbuf_v.at[pl.ds(0, ROWS_PT)],
                  dego_hbm.at[pl.ds(cid * NP + row0, ROWS_PT)])
  pltpu.sync_copy(degi_sh.at[pl.ds(row0, ROWS_PT)],
                  zbuf_v.at[pl.ds(0, ROWS_PT)])
  pltpu.sync_copy(zbuf_v.at[pl.ds(0, ROWS_PT)],
                  degi_hbm.at[pl.ds(cid * NP + row0, ROWS_PT)])


def _make_agg_kernel(d_feat, table_in_spmem=False):
  scratch = [
      pltpu.VMEM((CPP + NBUF, CH), jnp.int32),
      pltpu.VMEM((CPP, CH), jnp.int32),
      pltpu.VMEM((CH, d_feat), jnp.float32),
      pltpu.VMEM((CH, d_feat), jnp.float32),
      pltpu.VMEM_SHARED((NP, d_feat), jnp.float32),
      pltpu.SemaphoreType.DMA,
  ]
  if table_in_spmem:
    scratch.append(pltpu.VMEM_SHARED((NP, d_feat), jnp.float32))

  @functools.partial(
      pl.kernel,
      out_type=jax.ShapeDtypeStruct((NC, NP, d_feat), jnp.float32),
      mesh=_sc_mesh(),
      compiler_params=pltpu.CompilerParams(use_tc_tiling_on_sc=False),
      scratch_types=scratch,
  )
  def agg_kernel(h_hbm, src_hbm, dst_hbm, out_hbm,
                 src_v, dst_v, r0_v, r1_v, agg_sh, sem, *maybe_tab):
    bufs = [r0_v, r1_v]
    cid = lax.axis_index("c")
    sid = lax.axis_index("s")
    wid = sid * NC + cid
    row0 = sid * ROWS_PT
    def zbody(i, carry):
      for j in range(d_feat // 16):
        r0_v[i, pl.ds(j * 16, 16)] = jnp.zeros((16,), jnp.float32)
      return carry
    lax.fori_loop(0, CH, zbody, 0)
    nstep = (ROWS_PT + CH - 1) // CH
    for k in range(nstep):
      sz = CH if k < ROWS_PT // CH else ROWS_PT % CH
      pltpu.sync_copy(r0_v.at[pl.ds(0, sz)],
                      agg_sh.at[pl.ds(row0 + k * CH, sz)])
    if table_in_spmem:
      tab_sh = maybe_tab[0]
      for k in range(nstep):
        sz = CH if k < ROWS_PT // CH else ROWS_PT % CH
        pltpu.sync_copy(h_hbm.at[pl.ds(row0 + k * CH, sz)],
                        r1_v.at[pl.ds(0, sz)])
        pltpu.sync_copy(r1_v.at[pl.ds(0, sz)],
                        tab_sh.at[pl.ds(row0 + k * CH, sz)])
      gsrc = tab_sh
    else:
      gsrc = h_hbm
    for t in range(NBUF):
      for j in range(CH // 16):
        src_v[CPP + t, pl.ds(j * 16, 16)] = jnp.zeros((16,), jnp.int32)
    plsc.subcore_barrier()
    for p in range(NCH // CPP):
      pltpu.sync_copy(src_hbm.at[wid, pl.ds(p * CPP, CPP)],
                      src_v.at[pl.ds(0, CPP)])
      pltpu.sync_copy(dst_hbm.at[wid, pl.ds(p * CPP, CPP)], dst_v)
      for t in range(NBUF - 1):
        pltpu.async_copy(gsrc.at[src_v.at[t]], bufs[t], sem)
      def body(i, carry):
        j0 = NBUF * i
        for t in range(NBUF):
          pltpu.make_async_copy(gsrc.at[src_v.at[j0 + t]], bufs[t],
                                sem).wait()
          pltpu.async_copy(gsrc.at[src_v.at[j0 + t + NBUF - 1]],
                           bufs[(t - 1) % NBUF], sem)
          pltpu.sync_copy(bufs[t], agg_sh.at[dst_v.at[j0 + t]], add=True)
        return carry
      lax.fori_loop(0, CPP // NBUF, body, 0)
      for t in range(NBUF - 1):
        pltpu.make_async_copy(gsrc.at[src_v.at[CPP + t]],
                              bufs[(CPP + t) % NBUF], sem).wait()
    plsc.subcore_barrier()
    for k in range(nstep):
      sz = CH if k < ROWS_PT // CH else ROWS_PT % CH
      pltpu.sync_copy(agg_sh.at[pl.ds(row0 + k * CH, sz)],
                      r0_v.at[pl.ds(0, sz)])
      pltpu.sync_copy(r0_v.at[pl.ds(0, sz)],
                      out_hbm.at[cid, pl.ds(row0 + k * CH, sz)])
  return agg_kernel


_agg64 = _make_agg_kernel(64, table_in_spmem=True)

DH = 64


@functools.partial(
    pl.kernel,
    out_type=jax.ShapeDtypeStruct((NC, NP, DH), jnp.float32),
    mesh=_sc_mesh(),
    compiler_params=pltpu.CompilerParams(use_tc_tiling_on_sc=False),
    scratch_types=[
        pltpu.VMEM((CPP + NBUF, CH), jnp.int32),
        pltpu.VMEM((CPP, CH), jnp.int32),
        pltpu.VMEM((CH, DH), jnp.float32),
        pltpu.VMEM((CH, DH), jnp.float32),
        pltpu.VMEM_SHARED((NP, DH), jnp.float32),
        pltpu.VMEM_SHARED((NP, DH), jnp.float32),
        pltpu.SemaphoreType.DMA,
    ],
)
def _agg128_cs(h_hbm, src_hbm, dst_hbm, out_hbm,
               src_v, dst_v, r0_v, r1_v, agg_sh, tab_sh, sem):
  bufs = [r0_v, r1_v]
  cid = lax.axis_index("c")
  sid = lax.axis_index("s")
  row0 = sid * ROWS_PT
  def zbody(i, carry):
    for j in range(DH // 16):
      r0_v[i, pl.ds(j * 16, 16)] = jnp.zeros((16,), jnp.float32)
    return carry
  lax.fori_loop(0, CH, zbody, 0)
  nstep = (ROWS_PT + CH - 1) // CH
  for k in range(nstep):
    sz = CH if k < ROWS_PT // CH else ROWS_PT % CH
    pltpu.sync_copy(r0_v.at[pl.ds(0, sz)],
                    agg_sh.at[pl.ds(row0 + k * CH, sz)])
    pltpu.sync_copy(h_hbm.at[cid, pl.ds(row0 + k * CH, sz)],
                    r1_v.at[pl.ds(0, sz)])
    pltpu.sync_copy(r1_v.at[pl.ds(0, sz)],
                    tab_sh.at[pl.ds(row0 + k * CH, sz)])
  for t in range(NBUF):
    for j in range(CH // 16):
      src_v[CPP + t, pl.ds(j * 16, 16)] = jnp.zeros((16,), jnp.int32)
  plsc.subcore_barrier()
  for w in range(2):
    for p in range(NCH // CPP):
      pltpu.sync_copy(src_hbm.at[2 * sid + w, pl.ds(p * CPP, CPP)],
                      src_v.at[pl.ds(0, CPP)])
      pltpu.sync_copy(dst_hbm.at[2 * sid + w, pl.ds(p * CPP, CPP)], dst_v)
      for t in range(NBUF - 1):
        pltpu.async_copy(tab_sh.at[src_v.at[t]], bufs[t], sem)
      def body(i, carry):
        j0 = NBUF * i
        for t in range(NBUF):
          pltpu.make_async_copy(tab_sh.at[src_v.at[j0 + t]], bufs[t],
                                sem).wait()
          pltpu.async_copy(tab_sh.at[src_v.at[j0 + t + NBUF - 1]],
                           bufs[(t - 1) % NBUF], sem)
          pltpu.sync_copy(bufs[t], agg_sh.at[dst_v.at[j0 + t]], add=True)
        return carry
      lax.fori_loop(0, CPP // NBUF, body, 0)
      for t in range(NBUF - 1):
        pltpu.make_async_copy(tab_sh.at[src_v.at[CPP + t]],
                              bufs[(CPP + t) % NBUF], sem).wait()
  plsc.subcore_barrier()
  for k in range(nstep):
    sz = CH if k < ROWS_PT // CH else ROWS_PT % CH
    pltpu.sync_copy(agg_sh.at[pl.ds(row0 + k * CH, sz)],
                    r0_v.at[pl.ds(0, sz)])
    pltpu.sync_copy(r0_v.at[pl.ds(0, sz)],
                    out_hbm.at[cid, pl.ds(row0 + k * CH, sz)])


def _mm_body(x_ref, w_ref, o_ref):
  o_ref[...] = jnp.dot(x_ref[...], w_ref[...],
                       preferred_element_type=jnp.float32)


def _tc_matmul(x, w):
  return pl.pallas_call(
      _mm_body,
      out_shape=jax.ShapeDtypeStruct((x.shape[0], w.shape[1]), jnp.float32),
  )(x, w)


def _norm_body(hraw_ref, dego_ref, degi_ref, h1_ref, no_ref, ni_ref):
  deg_out = dego_ref[0, :] + dego_ref[1, :]
  deg_in = degi_ref[0, :] + degi_ref[1, :]
  norm_out = lax.rsqrt(jnp.maximum(deg_out, 1.0))
  norm_in = lax.rsqrt(jnp.maximum(deg_in, 1.0))
  no_ref[...] = jnp.broadcast_to(norm_out[:, None], (NP, 128))
  ni_ref[...] = jnp.broadcast_to(norm_in[:, None], (NP, 128))
  h1 = hraw_ref[...] * norm_out[:, None]
  h1_ref[0] = h1[:, :DH]
  h1_ref[1] = h1[:, DH:]


def _tc_norm_scale(hraw, dego, degi):
  return pl.pallas_call(
      _norm_body,
      out_shape=(
          jax.ShapeDtypeStruct((NC, NP, DH), jnp.float32),
          jax.ShapeDtypeStruct((NP, 128), jnp.float32),
          jax.ShapeDtypeStruct((NP, 128), jnp.float32),
      ),
  )(hraw, dego, degi)


def _mid_body(aggp_ref, ni_ref, no_ref, b1_ref, w2_ref, o_ref):
  agg = jnp.concatenate([aggp_ref[0], aggp_ref[1]], axis=1)
  h2 = jnp.maximum(agg * ni_ref[...] + b1_ref[...], 0.0)
  o_ref[...] = jnp.dot(h2 * no_ref[...], w2_ref[...],
                       preferred_element_type=jnp.float32)


def _tc_mid(aggp, ni, no, b1, w2):
  return pl.pallas_call(
      _mid_body,
      out_shape=jax.ShapeDtypeStruct((NP, w2.shape[1]), jnp.float32),
  )(aggp, ni, no, b1, w2)


def _final_body(aggp_ref, ni_ref, b2_ref, o_ref):
  agg = aggp_ref[0] + aggp_ref[1]
  o_ref[...] = agg * ni_ref[...] + b2_ref[...]


def _tc_final(aggp, ni, b2):
  return pl.pallas_call(
      _final_body,
      out_shape=jax.ShapeDtypeStruct((NP, b2.shape[1]), jnp.float32),
  )(aggp, ni, b2)


@jax.jit
def kernel(in_feat, edge_index, W1, b1, W2, b2):
  src = edge_index[0]
  dst = edge_index[1]
  pad = E_PAD - N_EDGES
  src_r = jnp.concatenate(
      [src, jnp.full((pad,), N_NODES, jnp.int32)]).reshape(NW, NCH, CH)
  trash = N_NODES + (jnp.arange(pad, dtype=jnp.int32) % (NP - N_NODES))
  dst_r = jnp.concatenate([dst, trash]).reshape(NW, NCH, CH)
  x_pad = jnp.pad(in_feat, ((0, NP - N_NODES), (0, 0)))

  dego, degi = _degree_kernel(src_r, dst_r)
  hraw = _tc_matmul(x_pad, W1)
  h1, no, ni = _tc_norm_scale(hraw, dego.reshape(NC, NP),
                              degi.reshape(NC, NP))
  aggp1 = _agg128_cs(h1, src_r, dst_r)
  h2b = _tc_mid(aggp1, ni, no, b1.reshape(1, 128), W2)
  aggp2 = _agg64(h2b, src_r, dst_r)
  out = _tc_final(aggp2, ni[:, :64], b2.reshape(1, 64))
  return out[:N_NODES]
